# baseline (speedup 1.0000x reference)
import jax
import jax.numpy as jnp
from jax import lax
from jax.experimental import pallas as pl
from jax.experimental.pallas import tpu as pltpu
from jax.experimental.pallas import tpu_sc as plsc

N = 10000
E = 320000
D = 128
H = 4
C = 128
B = 64

NP = 10240
BLK = 1024
NBLK = NP // BLK
DR = NP // 128
NC = 2
NS = 16
K = 64
STRIPE = NP // NS
DSTRIPE = 8
DTILES = DR // DSTRIPE
E_SL = E + N
CHUNKS = -(-E_SL // (NC * NS * K))
CHUNKS += CHUNKS % 2
PAIRS = CHUNKS // 2
E_PAD = NC * NS * K * CHUNKS


def _embed_tail(hb, asv_ref, adv_ref, hT_ref, ast_ref, adt_ref):
    s_rows, d_rows = [], []
    for h in range(H):
        hs = hb[:, h * C:(h + 1) * C]
        hT_ref[h] = hs
        s_rows.append(lax.dot_general(
            asv_ref[h:h + 1, :], hs, (((1,), (1,)), ((), ())),
            preferred_element_type=jnp.float32))
        d_rows.append(lax.dot_general(
            adv_ref[h:h + 1, :], hs, (((1,), (1,)), ((), ())),
            preferred_element_type=jnp.float32))
    ast_ref[...] = jnp.concatenate(s_rows, axis=0)
    adt_ref[...] = jnp.concatenate(d_rows, axis=0)


def _embed1_body(x_ref, W_ref, asv_ref, adv_ref, hT_ref, ast_ref, adt_ref):
    hb = jnp.dot(x_ref[...], W_ref[...], preferred_element_type=jnp.float32)
    _embed_tail(hb, asv_ref, adv_ref, hT_ref, ast_ref, adt_ref)


def _embed1(xp, W1, a_src, a_dst):
    return pl.pallas_call(
        _embed1_body,
        grid=(NBLK,),
        in_specs=[
            pl.BlockSpec((BLK, D), lambda j: (j, 0)),
            pl.BlockSpec((D, H * C), lambda j: (0, 0)),
            pl.BlockSpec((H, C), lambda j: (0, 0)),
            pl.BlockSpec((H, C), lambda j: (0, 0)),
        ],
        out_specs=[
            pl.BlockSpec((H, BLK, C), lambda j: (0, j, 0)),
            pl.BlockSpec((H, BLK), lambda j: (0, j)),
            pl.BlockSpec((H, BLK), lambda j: (0, j)),
        ],
        out_shape=[
            jax.ShapeDtypeStruct((H, NP, C), jnp.float32),
            jax.ShapeDtypeStruct((H, NP), jnp.float32),
            jax.ShapeDtypeStruct((H, NP), jnp.float32),
        ],
    )(xp, W1, a_src, a_dst)


def _combine(acc_ref, den_ref, b_ref):
    den_tot = jnp.maximum(den_ref[0] + den_ref[1], 1e-20)
    cols = []
    for h in range(H):
        t = acc_ref[h] + acc_ref[H + h]
        o = t / den_tot[:, h:h + 1] + b_ref[h:h + 1, :]
        cols.append(jnp.maximum(o, 0.0))
    return jnp.concatenate(cols, axis=1)


def _embed2_body(acc_ref, den_ref, b_ref, W_ref, asv_ref, adv_ref,
                 hT_ref, ast_ref, adt_ref):
    out1 = _combine(acc_ref, den_ref, b_ref)
    hb = jnp.dot(out1, W_ref[...], preferred_element_type=jnp.float32)
    _embed_tail(hb, asv_ref, adv_ref, hT_ref, ast_ref, adt_ref)


def _embed2(acc, den, b, W2, a_src, a_dst):
    return pl.pallas_call(
        _embed2_body,
        grid=(NBLK,),
        in_specs=[
            pl.BlockSpec((NC * H, BLK, C), lambda j: (0, j, 0)),
            pl.BlockSpec((NC, BLK, H), lambda j: (0, j, 0)),
            pl.BlockSpec((H, C), lambda j: (0, 0)),
            pl.BlockSpec((H * C, H * C), lambda j: (0, 0)),
            pl.BlockSpec((H, C), lambda j: (0, 0)),
            pl.BlockSpec((H, C), lambda j: (0, 0)),
        ],
        out_specs=[
            pl.BlockSpec((H, BLK, C), lambda j: (0, j, 0)),
            pl.BlockSpec((H, BLK), lambda j: (0, j)),
            pl.BlockSpec((H, BLK), lambda j: (0, j)),
        ],
        out_shape=[
            jax.ShapeDtypeStruct((H, NP, C), jnp.float32),
            jax.ShapeDtypeStruct((H, NP), jnp.float32),
            jax.ShapeDtypeStruct((H, NP), jnp.float32),
        ],
    )(acc, den, b, W2, a_src, a_dst)


def _sc_body(h_ref, as_ref, ad_ref, src_ref, dst_ref,
             out_ref, dout_ref,
             asrc_v, adst_v, idx80, den_v,
             srcq0, dstq0, idxq0, eeq0, srows0,
             srcq1, dstq1, idxq1, eeq1, srows1, scatq1,
             acc, dacc, semA, semB, ssemA, ssemB):
    c = lax.axis_index("c")
    s = lax.axis_index("s")
    wid = c * NS + s
    zero16 = jnp.zeros((16,), jnp.float32)
    lanes = lax.broadcasted_iota(jnp.int32, (16,), 0)
    bufs = ((srcq0, dstq0, idxq0, eeq0, srows0),
            (srcq1, dstq1, idxq1, eeq1, srows1))

    def zero_srows(i, carry):
        for j in range(C // 16):
            srows0[i, pl.ds(j * 16, 16)] = zero16
        return carry

    def zero_den(i, carry):
        for j in range(C // 16):
            den_v[i, pl.ds(j * 16, 16)] = zero16
        return carry

    for g in range(DR // 16):
        idx80[pl.ds(g * 16, 16)] = g * 16 + lanes

    for h in range(H):
        base = wid * (CHUNKS * K)

        def load_idx(ci, b):
            srcq, dstq, idxq, eeq, srows = b
            off = base + ci * K
            pltpu.sync_copy(src_ref.at[pl.ds(off, K)], srcq)
            pltpu.sync_copy(dst_ref.at[pl.ds(off, K)], dstq)

        def do_grp(b):
            srcq, dstq, idxq, eeq, srows = b

            def grp(g, carry2):
                s16 = srcq[pl.ds(g * 16, 16)]
                d16 = dstq[pl.ds(g * 16, 16)]
                idxq[pl.ds(g * 16, 16)] = s16 + (h * NP)
                e = (plsc.load_gather(asrc_v, [s16])
                     + plsc.load_gather(adst_v, [d16]))
                e = jnp.where(e >= 0.0, e, e * jnp.float32(0.2))
                ee = jnp.exp(e)
                eeq[pl.ds(g * 16, 16)] = ee
                row16 = lax.shift_right_logical(d16, 7)
                col16 = lax.bitwise_and(d16, 127)
                for li in range(16):
                    plsc.addupdate_scatter(den_v, [row16, col16], ee,
                                           mask=lanes == li)
                return carry2

            lax.fori_loop(0, K // 16, grp, 0)

        def do_scale(b):
            srcq, dstq, idxq, eeq, srows = b

            @plsc.parallel_loop(0, K // 16, unroll=2)
            def scale(g):
                ee16 = eeq[pl.ds(g * 16, 16)]
                for li in range(16):
                    w = ee16[li]
                    ei = g * 16 + li
                    for j in range(C // 16):
                        srows[ei, pl.ds(j * 16, 16)] = (
                            srows[ei, pl.ds(j * 16, 16)] * w)

        def gather_start(b, sem):
            pltpu.async_copy(h_ref.at[b[2]], b[4], sem)

        def gather_wait(b, sem):
            pltpu.make_async_copy(h_ref.at[b[2]], b[4], sem).wait()

        def scat_start(b, sem, iref=None):
            iref = b[1] if iref is None else iref
            pltpu.async_copy(b[4], acc.at[iref], sem, add=True)

        def scat_wait(b, sem, iref=None):
            iref = b[1] if iref is None else iref
            pltpu.make_async_copy(b[4], acc.at[iref], sem).wait()

        pltpu.sync_copy(as_ref.at[pl.ds(h * NP, NP)], asrc_v)
        pltpu.sync_copy(ad_ref.at[pl.ds(h * NP, NP)], adst_v)
        lax.fori_loop(0, K, zero_srows, 0)
        lax.fori_loop(0, DR, zero_den, 0)
        for k2 in range(STRIPE // K):
            pltpu.sync_copy(srows0, acc.at[pl.ds(s * STRIPE + k2 * K, K)])
        @pl.when(s < DTILES)
        def _():
            pltpu.sync_copy(srows0.at[pl.ds(0, DSTRIPE)],
                            dacc.at[pl.ds(s * DSTRIPE, DSTRIPE)])
        plsc.subcore_barrier()

        load_idx(0, bufs[0])
        do_grp(bufs[0])
        gather_start(bufs[0], semA)

        def pair(j, carry):
            load_idx(2 * j + 1, bufs[1])
            do_grp(bufs[1])
            @pl.when(j > 0)
            def _():
                scat_wait(bufs[1], ssemB, scatq1)
            gather_start(bufs[1], semB)
            gather_wait(bufs[0], semA)
            do_scale(bufs[0])
            scat_start(bufs[0], ssemA)
            gather_wait(bufs[1], semB)
            do_scale(bufs[1])
            for g in range(K // 16):
                scatq1[pl.ds(g * 16, 16)] = dstq1[pl.ds(g * 16, 16)]
            scat_start(bufs[1], ssemB, scatq1)

            @pl.when(j < PAIRS - 1)
            def _():
                scat_wait(bufs[0], ssemA)
                load_idx(2 * j + 2, bufs[0])
                do_grp(bufs[0])
                gather_start(bufs[0], semA)
            return carry

        lax.fori_loop(0, PAIRS, pair, 0)
        scat_wait(bufs[0], ssemA)
        scat_wait(bufs[1], ssemB, scatq1)
        pltpu.sync_copy(den_v, dacc.at[idx80], add=True)
        plsc.subcore_barrier()
        out_base = (c * H + h) * NP + s * STRIPE
        pltpu.sync_copy(acc.at[pl.ds(s * STRIPE, STRIPE)],
                        out_ref.at[pl.ds(out_base, STRIPE)])
        @pl.when(s < DTILES)
        def _():
            dout_base = (c * H + h) * DR + s * DSTRIPE
            pltpu.sync_copy(dacc.at[pl.ds(s * DSTRIPE, DSTRIPE)],
                            dout_ref.at[pl.ds(dout_base, DSTRIPE)])
        plsc.subcore_barrier()


def _sc_agg(h_flat, as_flat, ad_flat, src, dst):
    mesh = plsc.VectorSubcoreMesh(core_axis_name="c", subcore_axis_name="s")
    f = pl.kernel(
        _sc_body,
        out_type=[
            jax.ShapeDtypeStruct((NC * H * NP, C), jnp.float32),
            jax.ShapeDtypeStruct((NC * H * DR, 128), jnp.float32),
        ],
        mesh=mesh,
        compiler_params=pltpu.CompilerParams(needs_layout_passes=False),
        scratch_types=(
            [pltpu.VMEM((NP,), jnp.float32),
             pltpu.VMEM((NP,), jnp.float32),
             pltpu.VMEM((DR,), jnp.int32),
             pltpu.VMEM((DR, 128), jnp.float32)]
            + [pltpu.VMEM((K,), jnp.int32),
               pltpu.VMEM((K,), jnp.int32),
               pltpu.VMEM((K,), jnp.int32),
               pltpu.VMEM((K,), jnp.float32),
               pltpu.VMEM((K, C), jnp.float32)]
            + [pltpu.VMEM((K,), jnp.int32),
               pltpu.VMEM((K,), jnp.int32),
               pltpu.VMEM((K,), jnp.int32),
               pltpu.VMEM((K,), jnp.float32),
               pltpu.VMEM((K, C), jnp.float32),
               pltpu.VMEM((K,), jnp.int32)]
            + [pltpu.VMEM_SHARED((NP, C), jnp.float32),
               pltpu.VMEM_SHARED((DR, 128), jnp.float32),
               pltpu.SemaphoreType.DMA,
               pltpu.SemaphoreType.DMA,
               pltpu.SemaphoreType.DMA,
               pltpu.SemaphoreType.DMA]
        ),
    )
    return f(h_flat, as_flat, ad_flat, src, dst)


def _final_body(acc_ref, den_ref, b_ref, batch_ref, prev_ref, x_ref,
                Wr_ref, br_ref, Wn_ref, bn_ref, Wc_ref, bc_ref,
                out_ref, pool_acc, cnt_acc, news_acc):
    j = pl.program_id(0)
    h2 = _combine(acc_ref, den_ref, b_ref)
    bt = batch_ref[...]
    onehot = (bt == lax.broadcasted_iota(jnp.int32, (BLK, B), 1)
              ).astype(jnp.float32)
    isf = (bt != prev_ref[...]).astype(jnp.float32)
    rootm = onehot * isf
    pc = lax.dot_general(onehot, h2, (((0,), (0,)), ((), ())),
                         preferred_element_type=jnp.float32)
    ones = jnp.ones((BLK, 128), jnp.float32)
    cc = lax.dot_general(onehot, ones, (((0,), (0,)), ((), ())),
                         preferred_element_type=jnp.float32)
    nc_ = lax.dot_general(rootm, x_ref[...], (((0,), (0,)), ((), ())),
                          preferred_element_type=jnp.float32)

    @pl.when(j == 0)
    def _():
        pool_acc[...] = pc
        cnt_acc[...] = cc
        news_acc[...] = nc_

    @pl.when(j > 0)
    def _():
        pool_acc[...] += pc
        cnt_acc[...] += cc
        news_acc[...] += nc_

    @pl.when(j == NBLK - 1)
    def _():
        pooled = pool_acc[...] / jnp.maximum(cnt_acc[...][:, 0:1], 1.0)
        hr = jnp.maximum(
            jnp.dot(pooled, Wr_ref[...], preferred_element_type=jnp.float32)
            + br_ref[...], 0.0)
        news = jnp.maximum(
            jnp.dot(news_acc[...], Wn_ref[...],
                    preferred_element_type=jnp.float32) + bn_ref[...], 0.0)
        z = (jnp.dot(hr, Wc_ref[0:C, :], preferred_element_type=jnp.float32)
             + jnp.dot(news, Wc_ref[C:2 * C, :],
                       preferred_element_type=jnp.float32)
             + bc_ref[...])
        out_ref[...] = 1.0 / (1.0 + jnp.exp(-z))


def _final(acc, den, b2, batch_p, prev_p, xp, Wr, br, Wn, bn, Wc, bc):
    return pl.pallas_call(
        _final_body,
        grid=(NBLK,),
        in_specs=[
            pl.BlockSpec((NC * H, BLK, C), lambda j: (0, j, 0)),
            pl.BlockSpec((NC, BLK, H), lambda j: (0, j, 0)),
            pl.BlockSpec((H, C), lambda j: (0, 0)),
            pl.BlockSpec((BLK, 1), lambda j: (j, 0)),
            pl.BlockSpec((BLK, 1), lambda j: (j, 0)),
            pl.BlockSpec((BLK, D), lambda j: (j, 0)),
            pl.BlockSpec((H * C, C), lambda j: (0, 0)),
            pl.BlockSpec((1, C), lambda j: (0, 0)),
            pl.BlockSpec((D, C), lambda j: (0, 0)),
            pl.BlockSpec((1, C), lambda j: (0, 0)),
            pl.BlockSpec((2 * C, 1), lambda j: (0, 0)),
            pl.BlockSpec((1, 1), lambda j: (0, 0)),
        ],
        out_specs=pl.BlockSpec((B, 1), lambda j: (0, 0)),
        out_shape=jax.ShapeDtypeStruct((B, 1), jnp.float32),
        scratch_shapes=[
            pltpu.VMEM((B, H * C), jnp.float32),
            pltpu.VMEM((B, 128), jnp.float32),
            pltpu.VMEM((B, D), jnp.float32),
        ],
    )(acc, den, b2, batch_p, prev_p, xp, Wr, br, Wn, bn, Wc, bc)


def _den_layout(den_flat):
    return den_flat.reshape(NC, H, NP).transpose(0, 2, 1)


def kernel(x, edge_index, batch, W1, a_src1, a_dst1, b1,
           W2, a_src2, a_dst2, b2, Wr, br, Wn, bn, Wc, bc):
    xp = jnp.pad(x, ((0, NP - N), (0, 0)))
    loops = jnp.arange(N, dtype=jnp.int32)
    src = jnp.concatenate([edge_index[0], loops])
    dst = jnp.concatenate([edge_index[1], loops])
    src = jnp.pad(src, (0, E_PAD - E_SL), constant_values=N)
    dst = jnp.pad(dst, (0, E_PAD - E_SL), constant_values=N)
    batch_p = jnp.pad(batch, (0, NP - N), constant_values=B)[:, None]
    prev = jnp.concatenate([jnp.full((1,), -1, jnp.int32), batch[:-1]])
    prev_p = jnp.pad(prev, (0, NP - N), constant_values=-2)[:, None]

    hT1, ast1, adt1 = _embed1(xp, W1, a_src1, a_dst1)
    acc1, den1 = _sc_agg(hT1.reshape(H * NP, C), ast1.reshape(H * NP),
                         adt1.reshape(H * NP), src, dst)
    hT2, ast2, adt2 = _embed2(acc1.reshape(NC * H, NP, C), _den_layout(den1),
                              b1.reshape(H, C), W2, a_src2, a_dst2)
    acc2, den2 = _sc_agg(hT2.reshape(H * NP, C), ast2.reshape(H * NP),
                         adt2.reshape(H * NP), src, dst)
    return _final(acc2.reshape(NC * H, NP, C), _den_layout(den2),
                  b2.reshape(H, C), batch_p, prev_p, xp, Wr, br.reshape(1, C),
                  Wn, bn.reshape(1, C), Wc, bc.reshape(1, 1))


# speedup vs baseline: 1.1497x; 1.1497x over previous
import jax
import jax.numpy as jnp
from jax import lax
from jax.experimental import pallas as pl
from jax.experimental.pallas import tpu as pltpu
from jax.experimental.pallas import tpu_sc as plsc

N = 10000
E = 320000
D = 128
H = 4
C = 128
B = 64

NP = 10240
BLK = 1024
NBLK = NP // BLK
DR = NP // 128
NC = 2
NS = 16
K = 96
STRIPE = NP // NS
DSTRIPE = 8
DTILES = DR // DSTRIPE
E_SL = E + N
CHUNKS = -(-E_SL // (NC * NS * K))
CHUNKS += CHUNKS % 2
PAIRS = CHUNKS // 2
E_PAD = NC * NS * K * CHUNKS


HMASK = -65536


def _embed_tail(hb, asv_ref, adv_ref, hT_ref, pk_ref):
    s_rows, d_rows = [], []
    for h in range(H):
        hs = hb[:, h * C:(h + 1) * C]
        hT_ref[h] = hs
        s_rows.append(lax.dot_general(
            asv_ref[h:h + 1, :], hs, (((1,), (1,)), ((), ())),
            preferred_element_type=jnp.float32))
        d_rows.append(lax.dot_general(
            adv_ref[h:h + 1, :], hs, (((1,), (1,)), ((), ())),
            preferred_element_type=jnp.float32))
    asb = lax.bitcast_convert_type(jnp.concatenate(s_rows, axis=0), jnp.int32)
    adb = lax.bitcast_convert_type(jnp.concatenate(d_rows, axis=0), jnp.int32)
    pk_ref[...] = lax.bitwise_or(
        lax.bitwise_and(asb, jnp.int32(HMASK)), lax.shift_right_logical(adb, 16))


def _embed1_body(x_ref, W_ref, asv_ref, adv_ref, hT_ref, pk_ref):
    hb = jnp.dot(x_ref[...], W_ref[...], preferred_element_type=jnp.float32)
    _embed_tail(hb, asv_ref, adv_ref, hT_ref, pk_ref)


def _embed1(xp, W1, a_src, a_dst):
    return pl.pallas_call(
        _embed1_body,
        grid=(NBLK,),
        in_specs=[
            pl.BlockSpec((BLK, D), lambda j: (j, 0)),
            pl.BlockSpec((D, H * C), lambda j: (0, 0)),
            pl.BlockSpec((H, C), lambda j: (0, 0)),
            pl.BlockSpec((H, C), lambda j: (0, 0)),
        ],
        out_specs=[
            pl.BlockSpec((H, BLK, C), lambda j: (0, j, 0)),
            pl.BlockSpec((H, BLK), lambda j: (0, j)),
        ],
        out_shape=[
            jax.ShapeDtypeStruct((H, NP, C), jnp.float32),
            jax.ShapeDtypeStruct((H, NP), jnp.int32),
        ],
    )(xp, W1, a_src, a_dst)


def _combine(acc_ref, den_ref, b_ref):
    den_tot = jnp.maximum(den_ref[0] + den_ref[1], 1e-20)
    cols = []
    for h in range(H):
        t = acc_ref[h] + acc_ref[H + h]
        o = t / den_tot[:, h:h + 1] + b_ref[h:h + 1, :]
        cols.append(jnp.maximum(o, 0.0))
    return jnp.concatenate(cols, axis=1)


def _embed2_body(acc_ref, den_ref, b_ref, W_ref, asv_ref, adv_ref,
                 hT_ref, pk_ref):
    out1 = _combine(acc_ref, den_ref, b_ref)
    hb = jnp.dot(out1, W_ref[...], preferred_element_type=jnp.float32)
    _embed_tail(hb, asv_ref, adv_ref, hT_ref, pk_ref)


def _embed2(acc, den, b, W2, a_src, a_dst):
    return pl.pallas_call(
        _embed2_body,
        grid=(NBLK,),
        in_specs=[
            pl.BlockSpec((NC * H, BLK, C), lambda j: (0, j, 0)),
            pl.BlockSpec((NC, BLK, H), lambda j: (0, j, 0)),
            pl.BlockSpec((H, C), lambda j: (0, 0)),
            pl.BlockSpec((H * C, H * C), lambda j: (0, 0)),
            pl.BlockSpec((H, C), lambda j: (0, 0)),
            pl.BlockSpec((H, C), lambda j: (0, 0)),
        ],
        out_specs=[
            pl.BlockSpec((H, BLK, C), lambda j: (0, j, 0)),
            pl.BlockSpec((H, BLK), lambda j: (0, j)),
        ],
        out_shape=[
            jax.ShapeDtypeStruct((H, NP, C), jnp.float32),
            jax.ShapeDtypeStruct((H, NP), jnp.int32),
        ],
    )(acc, den, b, W2, a_src, a_dst)


def _sc_body(h_ref, pk_hbm, src_ref, dst_ref,
             out_ref, dout_ref,
             pk_v, idx80, den_v,
             srcq0, dstq0, idxq0, eeq0, srows0,
             srcq1, dstq1, idxq1, eeq1, srows1, scatq1,
             acc, dacc, semA, semB, ssemA, ssemB):
    c = lax.axis_index("c")
    s = lax.axis_index("s")
    wid = c * NS + s
    zero16 = jnp.zeros((16,), jnp.float32)
    lanes = lax.broadcasted_iota(jnp.int32, (16,), 0)
    bufs = ((srcq0, dstq0, idxq0, eeq0, srows0),
            (srcq1, dstq1, idxq1, eeq1, srows1))

    def zero_srows(i, carry):
        for j in range(C // 16):
            srows0[i, pl.ds(j * 16, 16)] = zero16
        return carry

    def zero_den(i, carry):
        for j in range(C // 16):
            den_v[i, pl.ds(j * 16, 16)] = zero16
        return carry

    for g in range(DR // 16):
        idx80[pl.ds(g * 16, 16)] = g * 16 + lanes

    for h in range(H):
        base = wid * (CHUNKS * K)

        def load_idx(ci, b):
            srcq, dstq, idxq, eeq, srows = b
            off = base + ci * K
            pltpu.sync_copy(src_ref.at[pl.ds(off, K)], srcq)
            pltpu.sync_copy(dst_ref.at[pl.ds(off, K)], dstq)

        def do_grp(b):
            srcq, dstq, idxq, eeq, srows = b

            def grp(g, carry2):
                s16 = srcq[pl.ds(g * 16, 16)]
                d16 = dstq[pl.ds(g * 16, 16)]
                idxq[pl.ds(g * 16, 16)] = s16 + (h * NP)
                ps = plsc.load_gather(pk_v, [s16])
                pd = plsc.load_gather(pk_v, [d16])
                asv = plsc.bitcast(lax.bitwise_and(ps, jnp.int32(HMASK)), jnp.float32)
                adv = plsc.bitcast(lax.shift_left(pd, 16), jnp.float32)
                e = asv + adv
                e = jnp.where(e >= 0.0, e, e * jnp.float32(0.2))
                ee = jnp.exp(e)
                eeq[pl.ds(g * 16, 16)] = ee
                row16 = lax.shift_right_logical(d16, 7)
                col16 = lax.bitwise_and(d16, 127)
                for li in range(16):
                    plsc.addupdate_scatter(den_v, [row16, col16], ee,
                                           mask=lanes == li)
                return carry2

            lax.fori_loop(0, K // 16, grp, 0)

        def do_scale(b):
            srcq, dstq, idxq, eeq, srows = b

            @plsc.parallel_loop(0, K // 16, unroll=2)
            def scale(g):
                ee16 = eeq[pl.ds(g * 16, 16)]
                for li in range(16):
                    w = ee16[li]
                    ei = g * 16 + li
                    for j in range(C // 16):
                        srows[ei, pl.ds(j * 16, 16)] = (
                            srows[ei, pl.ds(j * 16, 16)] * w)

        def gather_start(b, sem):
            pltpu.async_copy(h_ref.at[b[2]], b[4], sem)

        def gather_wait(b, sem):
            pltpu.make_async_copy(h_ref.at[b[2]], b[4], sem).wait()

        def scat_start(b, sem, iref=None):
            iref = b[1] if iref is None else iref
            pltpu.async_copy(b[4], acc.at[iref], sem, add=True)

        def scat_wait(b, sem, iref=None):
            iref = b[1] if iref is None else iref
            pltpu.make_async_copy(b[4], acc.at[iref], sem).wait()

        pltpu.sync_copy(pk_hbm.at[pl.ds(h * NP, NP)], pk_v)
        lax.fori_loop(0, K, zero_srows, 0)
        lax.fori_loop(0, DR, zero_den, 0)
        for k2 in range(STRIPE // 64):
            pltpu.sync_copy(srows0.at[pl.ds(0, 64)],
                            acc.at[pl.ds(s * STRIPE + k2 * 64, 64)])
        @pl.when(s < DTILES)
        def _():
            pltpu.sync_copy(srows0.at[pl.ds(0, DSTRIPE)],
                            dacc.at[pl.ds(s * DSTRIPE, DSTRIPE)])
        plsc.subcore_barrier()

        load_idx(0, bufs[0])
        do_grp(bufs[0])
        gather_start(bufs[0], semA)

        def pair(j, carry):
            load_idx(2 * j + 1, bufs[1])
            do_grp(bufs[1])
            @pl.when(j > 0)
            def _():
                scat_wait(bufs[1], ssemB, scatq1)
            gather_start(bufs[1], semB)
            gather_wait(bufs[0], semA)
            do_scale(bufs[0])
            scat_start(bufs[0], ssemA)
            gather_wait(bufs[1], semB)
            do_scale(bufs[1])
            for g in range(K // 16):
                scatq1[pl.ds(g * 16, 16)] = dstq1[pl.ds(g * 16, 16)]
            scat_start(bufs[1], ssemB, scatq1)

            @pl.when(j < PAIRS - 1)
            def _():
                scat_wait(bufs[0], ssemA)
                load_idx(2 * j + 2, bufs[0])
                do_grp(bufs[0])
                gather_start(bufs[0], semA)
            return carry

        lax.fori_loop(0, PAIRS, pair, 0)
        scat_wait(bufs[0], ssemA)
        scat_wait(bufs[1], ssemB, scatq1)
        pltpu.sync_copy(den_v, dacc.at[idx80], add=True)
        plsc.subcore_barrier()
        out_base = (c * H + h) * NP + s * STRIPE
        pltpu.sync_copy(acc.at[pl.ds(s * STRIPE, STRIPE)],
                        out_ref.at[pl.ds(out_base, STRIPE)])
        @pl.when(s < DTILES)
        def _():
            dout_base = (c * H + h) * DR + s * DSTRIPE
            pltpu.sync_copy(dacc.at[pl.ds(s * DSTRIPE, DSTRIPE)],
                            dout_ref.at[pl.ds(dout_base, DSTRIPE)])
        plsc.subcore_barrier()


def _sc_agg(h_flat, pk_flat, src, dst):
    mesh = plsc.VectorSubcoreMesh(core_axis_name="c", subcore_axis_name="s")
    f = pl.kernel(
        _sc_body,
        out_type=[
            jax.ShapeDtypeStruct((NC * H * NP, C), jnp.float32),
            jax.ShapeDtypeStruct((NC * H * DR, 128), jnp.float32),
        ],
        mesh=mesh,
        compiler_params=pltpu.CompilerParams(needs_layout_passes=False),
        scratch_types=(
            [pltpu.VMEM((NP,), jnp.int32),
             pltpu.VMEM((DR,), jnp.int32),
             pltpu.VMEM((DR, 128), jnp.float32)]
            + [pltpu.VMEM((K,), jnp.int32),
               pltpu.VMEM((K,), jnp.int32),
               pltpu.VMEM((K,), jnp.int32),
               pltpu.VMEM((K,), jnp.float32),
               pltpu.VMEM((K, C), jnp.float32)]
            + [pltpu.VMEM((K,), jnp.int32),
               pltpu.VMEM((K,), jnp.int32),
               pltpu.VMEM((K,), jnp.int32),
               pltpu.VMEM((K,), jnp.float32),
               pltpu.VMEM((K, C), jnp.float32),
               pltpu.VMEM((K,), jnp.int32)]
            + [pltpu.VMEM_SHARED((NP, C), jnp.float32),
               pltpu.VMEM_SHARED((DR, 128), jnp.float32),
               pltpu.SemaphoreType.DMA,
               pltpu.SemaphoreType.DMA,
               pltpu.SemaphoreType.DMA,
               pltpu.SemaphoreType.DMA]
        ),
    )
    return f(h_flat, pk_flat, src, dst)


def _final_body(acc_ref, den_ref, b_ref, batch_ref, prev_ref, x_ref,
                Wr_ref, br_ref, Wn_ref, bn_ref, Wc_ref, bc_ref,
                out_ref, pool_acc, cnt_acc, news_acc):
    j = pl.program_id(0)
    h2 = _combine(acc_ref, den_ref, b_ref)
    bt = batch_ref[...]
    onehot = (bt == lax.broadcasted_iota(jnp.int32, (BLK, B), 1)
              ).astype(jnp.float32)
    isf = (bt != prev_ref[...]).astype(jnp.float32)
    rootm = onehot * isf
    pc = lax.dot_general(onehot, h2, (((0,), (0,)), ((), ())),
                         preferred_element_type=jnp.float32)
    ones = jnp.ones((BLK, 128), jnp.float32)
    cc = lax.dot_general(onehot, ones, (((0,), (0,)), ((), ())),
                         preferred_element_type=jnp.float32)
    nc_ = lax.dot_general(rootm, x_ref[...], (((0,), (0,)), ((), ())),
                          preferred_element_type=jnp.float32)

    @pl.when(j == 0)
    def _():
        pool_acc[...] = pc
        cnt_acc[...] = cc
        news_acc[...] = nc_

    @pl.when(j > 0)
    def _():
        pool_acc[...] += pc
        cnt_acc[...] += cc
        news_acc[...] += nc_

    @pl.when(j == NBLK - 1)
    def _():
        pooled = pool_acc[...] / jnp.maximum(cnt_acc[...][:, 0:1], 1.0)
        hr = jnp.maximum(
            jnp.dot(pooled, Wr_ref[...], preferred_element_type=jnp.float32)
            + br_ref[...], 0.0)
        news = jnp.maximum(
            jnp.dot(news_acc[...], Wn_ref[...],
                    preferred_element_type=jnp.float32) + bn_ref[...], 0.0)
        z = (jnp.dot(hr, Wc_ref[0:C, :], preferred_element_type=jnp.float32)
             + jnp.dot(news, Wc_ref[C:2 * C, :],
                       preferred_element_type=jnp.float32)
             + bc_ref[...])
        out_ref[...] = 1.0 / (1.0 + jnp.exp(-z))


def _final(acc, den, b2, batch_p, prev_p, xp, Wr, br, Wn, bn, Wc, bc):
    return pl.pallas_call(
        _final_body,
        grid=(NBLK,),
        in_specs=[
            pl.BlockSpec((NC * H, BLK, C), lambda j: (0, j, 0)),
            pl.BlockSpec((NC, BLK, H), lambda j: (0, j, 0)),
            pl.BlockSpec((H, C), lambda j: (0, 0)),
            pl.BlockSpec((BLK, 1), lambda j: (j, 0)),
            pl.BlockSpec((BLK, 1), lambda j: (j, 0)),
            pl.BlockSpec((BLK, D), lambda j: (j, 0)),
            pl.BlockSpec((H * C, C), lambda j: (0, 0)),
            pl.BlockSpec((1, C), lambda j: (0, 0)),
            pl.BlockSpec((D, C), lambda j: (0, 0)),
            pl.BlockSpec((1, C), lambda j: (0, 0)),
            pl.BlockSpec((2 * C, 1), lambda j: (0, 0)),
            pl.BlockSpec((1, 1), lambda j: (0, 0)),
        ],
        out_specs=pl.BlockSpec((B, 1), lambda j: (0, 0)),
        out_shape=jax.ShapeDtypeStruct((B, 1), jnp.float32),
        scratch_shapes=[
            pltpu.VMEM((B, H * C), jnp.float32),
            pltpu.VMEM((B, 128), jnp.float32),
            pltpu.VMEM((B, D), jnp.float32),
        ],
    )(acc, den, b2, batch_p, prev_p, xp, Wr, br, Wn, bn, Wc, bc)


def _den_layout(den_flat):
    return den_flat.reshape(NC, H, NP).transpose(0, 2, 1)


def kernel(x, edge_index, batch, W1, a_src1, a_dst1, b1,
           W2, a_src2, a_dst2, b2, Wr, br, Wn, bn, Wc, bc):
    xp = jnp.pad(x, ((0, NP - N), (0, 0)))
    loops = jnp.arange(N, dtype=jnp.int32)
    src = jnp.concatenate([edge_index[0], loops])
    dst = jnp.concatenate([edge_index[1], loops])
    src = jnp.pad(src, (0, E_PAD - E_SL), constant_values=N)
    dst = jnp.pad(dst, (0, E_PAD - E_SL), constant_values=N)
    batch_p = jnp.pad(batch, (0, NP - N), constant_values=B)[:, None]
    prev = jnp.concatenate([jnp.full((1,), -1, jnp.int32), batch[:-1]])
    prev_p = jnp.pad(prev, (0, NP - N), constant_values=-2)[:, None]

    hT1, pk1 = _embed1(xp, W1, a_src1, a_dst1)
    acc1, den1 = _sc_agg(hT1.reshape(H * NP, C), pk1.reshape(H * NP),
                         src, dst)
    hT2, pk2 = _embed2(acc1.reshape(NC * H, NP, C), _den_layout(den1),
                       b1.reshape(H, C), W2, a_src2, a_dst2)
    acc2, den2 = _sc_agg(hT2.reshape(H * NP, C), pk2.reshape(H * NP),
                         src, dst)
    return _final(acc2.reshape(NC * H, NP, C), _den_layout(den2),
                  b2.reshape(H, C), batch_p, prev_p, xp, Wr, br.reshape(1, C),
                  Wn, bn.reshape(1, C), Wc, bc.reshape(1, 1))


# speedup vs baseline: 1.3532x; 1.1770x over previous
import jax
import jax.numpy as jnp
from jax import lax
from jax.experimental import pallas as pl
from jax.experimental.pallas import tpu as pltpu
from jax.experimental.pallas import tpu_sc as plsc

N = 10000
E = 320000
D = 128
H = 4
C = 128
B = 64

NP = 10240
BLK = 1024
NBLK = NP // BLK
DR = NP // 128
NC = 2
NS = 16
K = 96
STRIPE = NP // NS
DSTRIPE = 8
DTILES = DR // DSTRIPE
E_SL = E + N
CHUNKS = -(-E_SL // (NC * NS * K))
CHUNKS += CHUNKS % 2
PAIRS = CHUNKS // 2
E_PAD = NC * NS * K * CHUNKS


HMASK = -65536


def _embed_tail(hb, asv_ref, adv_ref, hT_ref, pk_ref):
    s_rows, d_rows = [], []
    for h in range(H):
        hs = hb[:, h * C:(h + 1) * C]
        hT_ref[h] = hs
        s_rows.append(lax.dot_general(
            asv_ref[h:h + 1, :], hs, (((1,), (1,)), ((), ())),
            preferred_element_type=jnp.float32))
        d_rows.append(lax.dot_general(
            adv_ref[h:h + 1, :], hs, (((1,), (1,)), ((), ())),
            preferred_element_type=jnp.float32))
    asb = lax.bitcast_convert_type(jnp.concatenate(s_rows, axis=0), jnp.int32)
    adb = lax.bitcast_convert_type(jnp.concatenate(d_rows, axis=0), jnp.int32)
    pk_ref[...] = lax.bitwise_or(
        lax.bitwise_and(asb, jnp.int32(HMASK)), lax.shift_right_logical(adb, 16))


def _embed1_body(x_ref, W_ref, asv_ref, adv_ref, hT_ref, pk_ref):
    hb = jnp.dot(x_ref[...], W_ref[...], preferred_element_type=jnp.float32)
    _embed_tail(hb, asv_ref, adv_ref, hT_ref, pk_ref)


def _embed1(xp, W1, a_src, a_dst):
    return pl.pallas_call(
        _embed1_body,
        grid=(NBLK,),
        in_specs=[
            pl.BlockSpec((BLK, D), lambda j: (j, 0)),
            pl.BlockSpec((D, H * C), lambda j: (0, 0)),
            pl.BlockSpec((H, C), lambda j: (0, 0)),
            pl.BlockSpec((H, C), lambda j: (0, 0)),
        ],
        out_specs=[
            pl.BlockSpec((H, BLK, C), lambda j: (0, j, 0)),
            pl.BlockSpec((H, BLK), lambda j: (0, j)),
        ],
        out_shape=[
            jax.ShapeDtypeStruct((H, NP, C), jnp.float32),
            jax.ShapeDtypeStruct((H, NP), jnp.int32),
        ],
    )(xp, W1, a_src, a_dst)


def _combine(acc_ref, den_ref, b_ref):
    den_tot = jnp.maximum(den_ref[0] + den_ref[1], 1e-20)
    cols = []
    for h in range(H):
        t = acc_ref[h] + acc_ref[H + h]
        o = t / den_tot[:, h:h + 1] + b_ref[h:h + 1, :]
        cols.append(jnp.maximum(o, 0.0))
    return jnp.concatenate(cols, axis=1)


def _embed2_body(acc_ref, den_ref, b_ref, W_ref, asv_ref, adv_ref,
                 hT_ref, pk_ref):
    out1 = _combine(acc_ref, den_ref, b_ref)
    hb = jnp.dot(out1, W_ref[...], preferred_element_type=jnp.float32)
    _embed_tail(hb, asv_ref, adv_ref, hT_ref, pk_ref)


def _embed2(acc, den, b, W2, a_src, a_dst):
    return pl.pallas_call(
        _embed2_body,
        grid=(NBLK,),
        in_specs=[
            pl.BlockSpec((NC * H, BLK, C), lambda j: (0, j, 0)),
            pl.BlockSpec((NC, BLK, H), lambda j: (0, j, 0)),
            pl.BlockSpec((H, C), lambda j: (0, 0)),
            pl.BlockSpec((H * C, H * C), lambda j: (0, 0)),
            pl.BlockSpec((H, C), lambda j: (0, 0)),
            pl.BlockSpec((H, C), lambda j: (0, 0)),
        ],
        out_specs=[
            pl.BlockSpec((H, BLK, C), lambda j: (0, j, 0)),
            pl.BlockSpec((H, BLK), lambda j: (0, j)),
        ],
        out_shape=[
            jax.ShapeDtypeStruct((H, NP, C), jnp.float32),
            jax.ShapeDtypeStruct((H, NP), jnp.int32),
        ],
    )(acc, den, b, W2, a_src, a_dst)


def _sc_body(h_ref, pk_hbm, sd_ref,
             out_ref, dout_ref,
             pk_v, idx80, den_v,
             sdq0, idxq0, eeq0, srows0, scatq0,
             sdq1, idxq1, eeq1, srows1, scatq1,
             acc, dacc, semA, semB, ssemA, ssemB, siA, siB):
    c = lax.axis_index("c")
    s = lax.axis_index("s")
    wid = c * NS + s
    zero16 = jnp.zeros((16,), jnp.float32)
    lanes = lax.broadcasted_iota(jnp.int32, (16,), 0)
    bufs = ((sdq0, idxq0, eeq0, srows0, scatq0, siA),
            (sdq1, idxq1, eeq1, srows1, scatq1, siB))

    def zero_srows(i, carry):
        for j in range(C // 16):
            srows0[i, pl.ds(j * 16, 16)] = zero16
        return carry

    def zero_den(i, carry):
        for j in range(C // 16):
            den_v[i, pl.ds(j * 16, 16)] = zero16
        return carry

    for g in range(DR // 16):
        idx80[pl.ds(g * 16, 16)] = g * 16 + lanes

    for h in range(H):
        cbase = wid * CHUNKS

        def idx_start(ci, b):
            pltpu.async_copy(sd_ref.at[pl.ds((cbase + ci) * 2 * K, 2 * K)],
                             b[0], b[5])

        def idx_wait(ci, b):
            pltpu.make_async_copy(
                sd_ref.at[pl.ds((cbase + ci) * 2 * K, 2 * K)],
                b[0], b[5]).wait()

        def do_grp(b):
            sdq, idxq, eeq, srows, scatq, si = b

            def grp(g, carry2):
                s16 = sdq[pl.ds(g * 16, 16)]
                d16 = sdq[pl.ds(K + g * 16, 16)]
                idxq[pl.ds(g * 16, 16)] = s16 + (h * NP)
                ps = plsc.load_gather(pk_v, [s16])
                pd = plsc.load_gather(pk_v, [d16])
                asv = plsc.bitcast(
                    lax.bitwise_and(ps, jnp.int32(HMASK)), jnp.float32)
                adv = plsc.bitcast(lax.shift_left(pd, 16), jnp.float32)
                e = asv + adv
                e = jnp.where(e >= 0.0, e, e * jnp.float32(0.2))
                ee = jnp.exp(e)
                eeq[pl.ds(g * 16, 16)] = ee
                row16 = lax.shift_right_logical(d16, 7)
                col16 = lax.bitwise_and(d16, 127)
                for li in range(16):
                    plsc.addupdate_scatter(den_v, [row16, col16], ee,
                                           mask=lanes == li)
                return carry2

            lax.fori_loop(0, K // 16, grp, 0)

        def do_scale(b):
            sdq, idxq, eeq, srows, scatq, si = b

            @plsc.parallel_loop(0, K // 16, unroll=2)
            def scale(g):
                ee16 = eeq[pl.ds(g * 16, 16)]
                for li in range(16):
                    w = ee16[li]
                    ei = g * 16 + li
                    for j in range(C // 16):
                        srows[ei, pl.ds(j * 16, 16)] = (
                            srows[ei, pl.ds(j * 16, 16)] * w)

        def snap_scatq(b):
            sdq, idxq, eeq, srows, scatq, si = b
            for g in range(K // 16):
                scatq[pl.ds(g * 16, 16)] = sdq[pl.ds(K + g * 16, 16)]

        def gather_start(b, sem):
            pltpu.async_copy(h_ref.at[b[1]], b[3], sem)

        def gather_wait(b, sem):
            pltpu.make_async_copy(h_ref.at[b[1]], b[3], sem).wait()

        def scat_start(b, sem):
            pltpu.async_copy(b[3], acc.at[b[4]], sem, add=True)

        def scat_wait(b, sem):
            pltpu.make_async_copy(b[3], acc.at[b[4]], sem).wait()

        pltpu.sync_copy(pk_hbm.at[pl.ds(h * NP, NP)], pk_v)
        lax.fori_loop(0, K, zero_srows, 0)
        lax.fori_loop(0, DR, zero_den, 0)
        for k2 in range(STRIPE // 64):
            pltpu.sync_copy(srows0.at[pl.ds(0, 64)],
                            acc.at[pl.ds(s * STRIPE + k2 * 64, 64)])
        @pl.when(s < DTILES)
        def _():
            pltpu.sync_copy(srows0.at[pl.ds(0, DSTRIPE)],
                            dacc.at[pl.ds(s * DSTRIPE, DSTRIPE)])
        plsc.subcore_barrier()

        idx_start(0, bufs[0])
        idx_wait(0, bufs[0])
        do_grp(bufs[0])
        snap_scatq(bufs[0])
        gather_start(bufs[0], semA)
        idx_start(1, bufs[1])

        def pair(j, carry):
            idx_wait(2 * j + 1, bufs[1])
            do_grp(bufs[1])
            snap_scatq(bufs[1])
            @pl.when(j > 0)
            def _():
                scat_wait(bufs[1], ssemB)
            gather_start(bufs[1], semB)

            @pl.when(j < PAIRS - 1)
            def _():
                idx_start(2 * j + 2, bufs[0])

            gather_wait(bufs[0], semA)
            do_scale(bufs[0])
            scat_start(bufs[0], ssemA)
            gather_wait(bufs[1], semB)
            do_scale(bufs[1])
            scat_start(bufs[1], ssemB)

            @pl.when(j < PAIRS - 1)
            def _():
                scat_wait(bufs[0], ssemA)
                idx_wait(2 * j + 2, bufs[0])
                do_grp(bufs[0])
                snap_scatq(bufs[0])
                gather_start(bufs[0], semA)
                idx_start(2 * j + 3, bufs[1])
            return carry

        lax.fori_loop(0, PAIRS, pair, 0)
        scat_wait(bufs[0], ssemA)
        scat_wait(bufs[1], ssemB)
        pltpu.sync_copy(den_v, dacc.at[idx80], add=True)
        plsc.subcore_barrier()
        out_base = (c * H + h) * NP + s * STRIPE
        pltpu.sync_copy(acc.at[pl.ds(s * STRIPE, STRIPE)],
                        out_ref.at[pl.ds(out_base, STRIPE)])
        @pl.when(s < DTILES)
        def _():
            dout_base = (c * H + h) * DR + s * DSTRIPE
            pltpu.sync_copy(dacc.at[pl.ds(s * DSTRIPE, DSTRIPE)],
                            dout_ref.at[pl.ds(dout_base, DSTRIPE)])
        plsc.subcore_barrier()


def _sc_agg(h_flat, pk_flat, sd):
    mesh = plsc.VectorSubcoreMesh(core_axis_name="c", subcore_axis_name="s")
    f = pl.kernel(
        _sc_body,
        out_type=[
            jax.ShapeDtypeStruct((NC * H * NP, C), jnp.float32),
            jax.ShapeDtypeStruct((NC * H * DR, 128), jnp.float32),
        ],
        mesh=mesh,
        compiler_params=pltpu.CompilerParams(needs_layout_passes=False),
        scratch_types=(
            [pltpu.VMEM((NP,), jnp.int32),
             pltpu.VMEM((DR,), jnp.int32),
             pltpu.VMEM((DR, 128), jnp.float32)]
            + 2 * [pltpu.VMEM((2 * K,), jnp.int32),
                   pltpu.VMEM((K,), jnp.int32),
                   pltpu.VMEM((K,), jnp.float32),
                   pltpu.VMEM((K, C), jnp.float32),
                   pltpu.VMEM((K,), jnp.int32)]
            + [pltpu.VMEM_SHARED((NP, C), jnp.float32),
               pltpu.VMEM_SHARED((DR, 128), jnp.float32)]
            + 6 * [pltpu.SemaphoreType.DMA]
        ),
    )
    return f(h_flat, pk_flat, sd)


def _final_body(acc_ref, den_ref, b_ref, batch_ref, prev_ref, x_ref,
                Wr_ref, br_ref, Wn_ref, bn_ref, Wc_ref, bc_ref,
                out_ref, pool_acc, cnt_acc, news_acc):
    j = pl.program_id(0)
    h2 = _combine(acc_ref, den_ref, b_ref)
    bt = batch_ref[...]
    onehot = (bt == lax.broadcasted_iota(jnp.int32, (BLK, B), 1)
              ).astype(jnp.float32)
    isf = (bt != prev_ref[...]).astype(jnp.float32)
    rootm = onehot * isf
    pc = lax.dot_general(onehot, h2, (((0,), (0,)), ((), ())),
                         preferred_element_type=jnp.float32)
    ones = jnp.ones((BLK, 128), jnp.float32)
    cc = lax.dot_general(onehot, ones, (((0,), (0,)), ((), ())),
                         preferred_element_type=jnp.float32)
    nc_ = lax.dot_general(rootm, x_ref[...], (((0,), (0,)), ((), ())),
                          preferred_element_type=jnp.float32)

    @pl.when(j == 0)
    def _():
        pool_acc[...] = pc
        cnt_acc[...] = cc
        news_acc[...] = nc_

    @pl.when(j > 0)
    def _():
        pool_acc[...] += pc
        cnt_acc[...] += cc
        news_acc[...] += nc_

    @pl.when(j == NBLK - 1)
    def _():
        pooled = pool_acc[...] / jnp.maximum(cnt_acc[...][:, 0:1], 1.0)
        hr = jnp.maximum(
            jnp.dot(pooled, Wr_ref[...], preferred_element_type=jnp.float32)
            + br_ref[...], 0.0)
        news = jnp.maximum(
            jnp.dot(news_acc[...], Wn_ref[...],
                    preferred_element_type=jnp.float32) + bn_ref[...], 0.0)
        z = (jnp.dot(hr, Wc_ref[0:C, :], preferred_element_type=jnp.float32)
             + jnp.dot(news, Wc_ref[C:2 * C, :],
                       preferred_element_type=jnp.float32)
             + bc_ref[...])
        out_ref[...] = 1.0 / (1.0 + jnp.exp(-z))


def _final(acc, den, b2, batch_p, prev_p, xp, Wr, br, Wn, bn, Wc, bc):
    return pl.pallas_call(
        _final_body,
        grid=(NBLK,),
        in_specs=[
            pl.BlockSpec((NC * H, BLK, C), lambda j: (0, j, 0)),
            pl.BlockSpec((NC, BLK, H), lambda j: (0, j, 0)),
            pl.BlockSpec((H, C), lambda j: (0, 0)),
            pl.BlockSpec((BLK, 1), lambda j: (j, 0)),
            pl.BlockSpec((BLK, 1), lambda j: (j, 0)),
            pl.BlockSpec((BLK, D), lambda j: (j, 0)),
            pl.BlockSpec((H * C, C), lambda j: (0, 0)),
            pl.BlockSpec((1, C), lambda j: (0, 0)),
            pl.BlockSpec((D, C), lambda j: (0, 0)),
            pl.BlockSpec((1, C), lambda j: (0, 0)),
            pl.BlockSpec((2 * C, 1), lambda j: (0, 0)),
            pl.BlockSpec((1, 1), lambda j: (0, 0)),
        ],
        out_specs=pl.BlockSpec((B, 1), lambda j: (0, 0)),
        out_shape=jax.ShapeDtypeStruct((B, 1), jnp.float32),
        scratch_shapes=[
            pltpu.VMEM((B, H * C), jnp.float32),
            pltpu.VMEM((B, 128), jnp.float32),
            pltpu.VMEM((B, D), jnp.float32),
        ],
    )(acc, den, b2, batch_p, prev_p, xp, Wr, br, Wn, bn, Wc, bc)


def _den_layout(den_flat):
    return den_flat.reshape(NC, H, NP).transpose(0, 2, 1)


def kernel(x, edge_index, batch, W1, a_src1, a_dst1, b1,
           W2, a_src2, a_dst2, b2, Wr, br, Wn, bn, Wc, bc):
    xp = jnp.pad(x, ((0, NP - N), (0, 0)))
    loops = jnp.arange(N, dtype=jnp.int32)
    src = jnp.concatenate([edge_index[0], loops])
    dst = jnp.concatenate([edge_index[1], loops])
    src = jnp.pad(src, (0, E_PAD - E_SL), constant_values=N)
    dst = jnp.pad(dst, (0, E_PAD - E_SL), constant_values=N)
    batch_p = jnp.pad(batch, (0, NP - N), constant_values=B)[:, None]
    prev = jnp.concatenate([jnp.full((1,), -1, jnp.int32), batch[:-1]])
    prev_p = jnp.pad(prev, (0, NP - N), constant_values=-2)[:, None]

    sd = jnp.stack([src.reshape(-1, K), dst.reshape(-1, K)],
                   axis=1).reshape(-1)

    hT1, pk1 = _embed1(xp, W1, a_src1, a_dst1)
    acc1, den1 = _sc_agg(hT1.reshape(H * NP, C), pk1.reshape(H * NP), sd)
    hT2, pk2 = _embed2(acc1.reshape(NC * H, NP, C), _den_layout(den1),
                       b1.reshape(H, C), W2, a_src2, a_dst2)
    acc2, den2 = _sc_agg(hT2.reshape(H * NP, C), pk2.reshape(H * NP), sd)
    return _final(acc2.reshape(NC * H, NP, C), _den_layout(den2),
                  b2.reshape(H, C), batch_p, prev_p, xp, Wr, br.reshape(1, C),
                  Wn, bn.reshape(1, C), Wc, bc.reshape(1, 1))
